# baseline (speedup 1.0000x reference)
import functools

import jax
import jax.numpy as jnp
from jax import lax
from jax.experimental import pallas as pl
from jax.experimental.pallas import tpu as pltpu
from jax.experimental.pallas import tpu_sc as plsc

BATCH = 4096
EMBED_DIM = 32
NUM_CORES = 2
NUM_SUBCORES = 16
NUM_WORKERS = NUM_CORES * NUM_SUBCORES
B_PER_W = BATCH // NUM_WORKERS


def _make_lookup_kernel():
    mesh = plsc.VectorSubcoreMesh(core_axis_name="c", subcore_axis_name="s")
    out = jax.ShapeDtypeStruct((BATCH, EMBED_DIM), jnp.float32)

    @functools.partial(
        pl.kernel,
        mesh=mesh,
        out_type=(out, out, out),
        compiler_params=pltpu.CompilerParams(use_tc_tiling_on_sc=False),
        scratch_types=[
            pltpu.VMEM((B_PER_W,), jnp.int32),
            pltpu.VMEM((B_PER_W,), jnp.int32),
            pltpu.VMEM((B_PER_W,), jnp.int32),
            pltpu.VMEM((B_PER_W, EMBED_DIM), jnp.float32),
            pltpu.VMEM((B_PER_W, EMBED_DIM), jnp.float32),
            pltpu.VMEM((B_PER_W, EMBED_DIM), jnp.float32),
            pltpu.SemaphoreType.DMA,
        ],
    )
    def lookup(uid_hbm, iid_hbm, cid_hbm, wu_hbm, wi_hbm, wc_hbm,
               out_u, out_i, out_c,
               idx_u, idx_i, idx_c, rows_u, rows_i, rows_c, sem):
        wid = lax.axis_index("s") * NUM_CORES + lax.axis_index("c")
        base = wid * B_PER_W
        pltpu.sync_copy(uid_hbm.at[pl.ds(base, B_PER_W)], idx_u)
        pltpu.sync_copy(iid_hbm.at[pl.ds(base, B_PER_W)], idx_i)
        pltpu.sync_copy(cid_hbm.at[pl.ds(base, B_PER_W)], idx_c)
        cu = pltpu.async_copy(wu_hbm.at[idx_u], rows_u, sem)
        ci = pltpu.async_copy(wi_hbm.at[idx_i], rows_i, sem)
        cc = pltpu.async_copy(wc_hbm.at[idx_c], rows_c, sem)
        cu.wait()
        ci.wait()
        cc.wait()
        pltpu.sync_copy(rows_u, out_u.at[pl.ds(base, B_PER_W)])
        pltpu.sync_copy(rows_i, out_i.at[pl.ds(base, B_PER_W)])
        pltpu.sync_copy(rows_c, out_c.at[pl.ds(base, B_PER_W)])

    return lookup


_lookup = _make_lookup_kernel()


def kernel(user_id, item_id, category, W_user, W_item, W_category):
    return _lookup(
        user_id.astype(jnp.int32),
        item_id.astype(jnp.int32),
        category.astype(jnp.int32),
        W_user,
        W_item,
        W_category,
    )
